# baseline (speedup 1.0000x reference)
import numpy as np
import jax
import jax.numpy as jnp
from jax.experimental import pallas as pl
from jax.experimental.pallas import tpu as pltpu

_V_PATCH_NUMS = (1, 2, 3, 4, 5, 6, 8, 10, 13, 16)
_VOCAB = 8192
_CVAE = 32
_BETA = 0.25
_QRESI = 0.5
_SHARE = 4
_B, _H, _W = 64, 16, 16
_SN = len(_V_PATCH_NUMS)
_TN = 256
_VT = 512
_NVT = _VOCAB // _VT


def _area_matrix(in_size, out_size):
    M = np.zeros((out_size, in_size), dtype=np.float64)
    for i in range(out_size):
        s = (i * in_size) // out_size
        e = -((-(i + 1) * in_size) // out_size)
        M[i, s:e] = 1.0 / (e - s)
    return M.astype(np.float32)


def _cubic_w(x, a=-0.75):
    x = abs(x)
    if x <= 1.0:
        return (a + 2.0) * x ** 3 - (a + 3.0) * x ** 2 + 1.0
    elif x < 2.0:
        return a * x ** 3 - 5.0 * a * x ** 2 + 8.0 * a * x - 4.0 * a
    return 0.0


def _bicubic_matrix(in_size, out_size):
    M = np.zeros((out_size, in_size), dtype=np.float64)
    scale = in_size / out_size
    for i in range(out_size):
        src = (i + 0.5) * scale - 0.5
        f = np.floor(src)
        t = src - f
        for k in range(-1, 3):
            idx = int(np.clip(f + k, 0, in_size - 1))
            M[i, idx] += _cubic_w(t - k)
    return M.astype(np.float32)


_TICKS = np.linspace(1 / 3 / _SHARE, 1 - 1 / 3 / _SHARE, _SHARE)
_PHI_IDX = [int(np.argmin(np.abs(_TICKS - si / (_SN - 1))))
            for si in range(_SN)]


def _vq_block(rest_ref, embt_ref, emb_ref, out_ref):
    rest = rest_ref[...]
    x2 = jnp.sum(rest ** 2, axis=1, keepdims=True)

    def vt_body(t, carry):
        best, barg = carry
        et = embt_ref[t]
        e2 = jnp.sum(et ** 2, axis=0)[None, :]
        s = jnp.dot(rest, et)
        d = (x2 + e2) - 2.0 * s
        tmin = jnp.min(d, axis=1)
        targ = jnp.argmin(d, axis=1)
        upd = tmin < best
        barg = jnp.where(upd, targ + t * _VT, barg)
        best = jnp.where(upd, tmin, best)
        return best, barg

    best0 = jnp.full((_TN,), jnp.inf, jnp.float32)
    barg0 = jnp.zeros((_TN,), jnp.int32)
    barg = jax.lax.fori_loop(0, _NVT, vt_body, (best0, barg0))[1]

    def oh_body(t, hs):
        lane = jax.lax.broadcasted_iota(jnp.int32, (_TN, _VT), 1) + t * _VT
        oh = jnp.where(lane == barg[:, None],
                       jnp.float32(1.0), jnp.float32(0.0))
        return hs + jnp.dot(oh, emb_ref[t],
                            precision=jax.lax.Precision.HIGHEST)

    out_ref[...] = jax.lax.fori_loop(
        0, _NVT, oh_body, jnp.zeros((_TN, _CVAE), jnp.float32))


def _vq_lookup(rest_NC, embt3, emb3):
    n = rest_NC.shape[0]
    npad = -(-n // _TN) * _TN
    restp = jnp.pad(rest_NC, ((0, npad - n), (0, 0)))
    h = pl.pallas_call(
        _vq_block,
        grid=(npad // _TN,),
        in_specs=[
            pl.BlockSpec((_TN, _CVAE), lambda i: (i, 0)),
            pl.BlockSpec((_NVT, _CVAE, _VT), lambda i: (0, 0, 0)),
            pl.BlockSpec((_NVT, _VT, _CVAE), lambda i: (0, 0, 0)),
        ],
        out_specs=pl.BlockSpec((_TN, _CVAE), lambda i: (i, 0)),
        out_shape=jax.ShapeDtypeStruct((npad, _CVAE), jnp.float32),
    )(restp, embt3, emb3)
    return h[:n]


def _phi_apply(h, w, b):
    conv = jax.lax.conv_general_dilated(
        h, w, window_strides=(1, 1), padding='SAME',
        dimension_numbers=('NCHW', 'OIHW', 'NCHW')) + b[None, :, None, None]
    return h * (1.0 - _QRESI) + conv * _QRESI


def kernel(f_BChw, emb_weight, phi_w, phi_b):
    f = f_BChw
    Bn, C, Hh, Ww = f.shape
    f_no_grad = jax.lax.stop_gradient(f)
    f_rest = f_no_grad
    f_hat = jnp.zeros_like(f)

    ew = jax.lax.stop_gradient(emb_weight)
    embt3 = jnp.transpose(ew.T.reshape(_CVAE, _NVT, _VT), (1, 0, 2))
    emb3 = ew.reshape(_NVT, _VT, _CVAE)

    loss = jnp.zeros((), dtype=jnp.float32)
    for si, pn in enumerate(_V_PATCH_NUMS):
        if si != _SN - 1:
            A = jnp.asarray(_area_matrix(Hh, pn))
            interp = jnp.einsum('oh,bchw->bcow', A, f_rest)
            interp = jnp.einsum('pw,bcow->bcop', A, interp)
        else:
            interp = f_rest
        rest_NC = jnp.transpose(interp, (0, 2, 3, 1)).reshape(-1, C)
        if pn in (13, 16):
            d = (jnp.sum(rest_NC ** 2, axis=1, keepdims=True)
                 + jnp.sum(ew ** 2, axis=1)[None, :]
                 - 2.0 * (rest_NC @ ew.T))
            idx_N = jnp.argmin(d, axis=1)
            h_NC = jnp.take(emb_weight, idx_N, axis=0)
        else:
            h_NC = _vq_lookup(rest_NC, embt3, emb3)
        h = h_NC.reshape(Bn, pn, pn, C).transpose(0, 3, 1, 2)
        if si != _SN - 1:
            U = jnp.asarray(_bicubic_matrix(pn, Hh))
            h = jnp.einsum('oh,bchw->bcow', U, h)
            h = jnp.einsum('pw,bcow->bcop', U, h)
        h = _phi_apply(h, phi_w[_PHI_IDX[si]], phi_b[_PHI_IDX[si]])
        f_hat = f_hat + h
        f_rest = f_rest - h
        loss = (loss
                + _BETA * jnp.mean((jax.lax.stop_gradient(f_hat) - f) ** 2)
                + jnp.mean((f_hat - f_no_grad) ** 2))
    loss = loss / _SN
    f_hat_final = jax.lax.stop_gradient(f_hat) - f_no_grad + f
    return f_hat_final, loss
